# Initial kernel scaffold; baseline (speedup 1.0000x reference)
import jax
import jax.numpy as jnp
from jax.experimental import pallas as pl


def kernel(class_logits, box_regression, proposals):
    raise NotImplementedError("write your pallas kernel here")


# speedup vs baseline: 1.0313x; 1.0313x over previous
import math

import jax
import jax.numpy as jnp
from jax.experimental import pallas as pl
from jax.experimental.pallas import tpu as pltpu

N_PROP = 5000
NUM_CLASSES = 81
PRE_NMS_TOPK = 500
SCORE_THRESH = 0.05
NMS_THRESH = 0.5
DET_PER_IMG = 100
IMG_H = 800.0
IMG_W = 800.0
BBOX_XFORM_CLIP = math.log(1000.0 / 16.0)


def _softmax_decode_body(logits_ref, dx_ref, dy_ref, dw_ref, dh_ref, props_ref,
                         prob_ref, x1_ref, y1_ref, x2_ref, y2_ref):
    logits = logits_ref[...]
    m = jnp.max(logits, axis=1, keepdims=True)
    e = jnp.exp(logits - m)
    prob_ref[...] = e / jnp.sum(e, axis=1, keepdims=True)

    p = props_ref[...]
    widths = p[:, 2:3] - p[:, 0:1]
    heights = p[:, 3:4] - p[:, 1:2]
    ctr_x = p[:, 0:1] + 0.5 * widths
    ctr_y = p[:, 1:2] + 0.5 * heights

    dx = dx_ref[...] * 0.1
    dy = dy_ref[...] * 0.1
    dw = jnp.minimum(dw_ref[...] * 0.2, BBOX_XFORM_CLIP)
    dh = jnp.minimum(dh_ref[...] * 0.2, BBOX_XFORM_CLIP)

    pcx = dx * widths + ctr_x
    pcy = dy * heights + ctr_y
    pw = jnp.exp(dw) * widths
    ph = jnp.exp(dh) * heights

    x1_ref[...] = jnp.clip(pcx - 0.5 * pw, 0.0, IMG_W)
    y1_ref[...] = jnp.clip(pcy - 0.5 * ph, 0.0, IMG_H)
    x2_ref[...] = jnp.clip(pcx + 0.5 * pw, 0.0, IMG_W)
    y2_ref[...] = jnp.clip(pcy + 0.5 * ph, 0.0, IMG_H)


def _softmax_decode(class_logits, box_regression, proposals):
    rel = box_regression.reshape(N_PROP, NUM_CLASSES, 4)
    dx = rel[..., 0]
    dy = rel[..., 1]
    dw = rel[..., 2]
    dh = rel[..., 3]
    shp = jax.ShapeDtypeStruct((N_PROP, NUM_CLASSES), jnp.float32)
    return pl.pallas_call(
        _softmax_decode_body,
        out_shape=(shp, shp, shp, shp, shp),
    )(class_logits, dx, dy, dw, dh, proposals)


def _box_iou(a, b):
    area_a = (a[:, 2] - a[:, 0]) * (a[:, 3] - a[:, 1])
    area_b = (b[:, 2] - b[:, 0]) * (b[:, 3] - b[:, 1])
    lt = jnp.maximum(a[:, None, :2], b[None, :, :2])
    rb = jnp.minimum(a[:, None, 2:], b[None, :, 2:])
    wh = jnp.clip(rb - lt, 0.0, None)
    inter = wh[..., 0] * wh[..., 1]
    return inter / (area_a[:, None] + area_b[None, :] - inter + 1e-9)


def _nms_keep(iou, valid):
    n = iou.shape[0]
    idxs = jnp.arange(n)

    def step(keep, i):
        sup = (iou[i] > NMS_THRESH) & (idxs > i) & keep[i]
        return keep & (~sup), None

    keep, _ = jax.lax.scan(step, valid, idxs)
    return keep


def _per_class_nms(scores, boxes):
    vals, idx = jax.lax.top_k(scores, PRE_NMS_TOPK)
    b = boxes[idx]
    valid = vals > SCORE_THRESH
    iou = _box_iou(b, b)
    keep = _nms_keep(iou, valid)
    masked = jnp.where(keep, vals, -1.0)
    return masked, b


def kernel(class_logits, box_regression, proposals):
    prob, x1, y1, x2, y2 = _softmax_decode(class_logits, box_regression, proposals)
    boxes = jnp.stack([x1, y1, x2, y2], axis=-1)
    prob_c = prob[:, 1:].T
    boxes_c = jnp.transpose(boxes[:, 1:, :], (1, 0, 2))
    scores_all, boxes_all = jax.vmap(_per_class_nms)(prob_c, boxes_c)
    flat_scores = scores_all.reshape(-1)
    flat_boxes = boxes_all.reshape(-1, 4)
    top_vals, top_idx = jax.lax.top_k(flat_scores, DET_PER_IMG)
    det = jnp.concatenate([flat_boxes[top_idx], top_vals[:, None]], axis=1)
    return det


# speedup vs baseline: 4.4579x; 4.3227x over previous
import functools
import math

import jax
import jax.numpy as jnp
from jax.experimental import pallas as pl
from jax.experimental.pallas import tpu as pltpu

N_PROP = 5000
NUM_CLASSES = 81
NUM_FG = 80
K = 500
KP = 512
NW = KP // 16
SCORE_THRESH = 0.05
DET_PER_IMG = 100
IMG_H = 800.0
IMG_W = 800.0
BBOX_XFORM_CLIP = math.log(1000.0 / 16.0)
BIG = 2**30


def _softmax_decode_body(logits_ref, dx_ref, dy_ref, dw_ref, dh_ref, props_ref,
                         prob_ref, x1_ref, y1_ref, x2_ref, y2_ref):
    logits = logits_ref[...]
    m = jnp.max(logits, axis=1, keepdims=True)
    e = jnp.exp(logits - m)
    prob_ref[...] = e / jnp.sum(e, axis=1, keepdims=True)

    p = props_ref[...]
    widths = p[:, 2:3] - p[:, 0:1]
    heights = p[:, 3:4] - p[:, 1:2]
    ctr_x = p[:, 0:1] + 0.5 * widths
    ctr_y = p[:, 1:2] + 0.5 * heights

    dx = dx_ref[...] * 0.1
    dy = dy_ref[...] * 0.1
    dw = jnp.minimum(dw_ref[...] * 0.2, BBOX_XFORM_CLIP)
    dh = jnp.minimum(dh_ref[...] * 0.2, BBOX_XFORM_CLIP)

    pcx = dx * widths + ctr_x
    pcy = dy * heights + ctr_y
    pw = jnp.exp(dw) * widths
    ph = jnp.exp(dh) * heights

    x1_ref[...] = jnp.clip(pcx - 0.5 * pw, 0.0, IMG_W)
    y1_ref[...] = jnp.clip(pcy - 0.5 * ph, 0.0, IMG_H)
    x2_ref[...] = jnp.clip(pcx + 0.5 * pw, 0.0, IMG_W)
    y2_ref[...] = jnp.clip(pcy + 0.5 * ph, 0.0, IMG_H)


def _softmax_decode(class_logits, box_regression, proposals):
    rel = box_regression.reshape(N_PROP, NUM_CLASSES, 4)
    shp = jax.ShapeDtypeStruct((N_PROP, NUM_CLASSES), jnp.float32)
    return pl.pallas_call(
        _softmax_decode_body,
        out_shape=(shp, shp, shp, shp, shp),
    )(class_logits, rel[..., 0], rel[..., 1], rel[..., 2], rel[..., 3], proposals)


def _pack_matrix():
    j16 = jax.lax.broadcasted_iota(jnp.int32, (KP, NW), 0)
    wl = jax.lax.broadcasted_iota(jnp.int32, (KP, NW), 1)
    pw = jnp.int32(1) << (j16 % 16)
    return jnp.where((j16 // 16) == wl, pw, 0).astype(jnp.float32)


def _iou_pack_body(x1r_ref, y1r_ref, x2r_ref, y2r_ref,
                   x1c_ref, y1c_ref, x2c_ref, y2c_ref, sp_ref):
    x1r, y1r, x2r, y2r = x1r_ref[0], y1r_ref[0], x2r_ref[0], y2r_ref[0]
    x1c, y1c, x2c, y2c = x1c_ref[0], y1c_ref[0], x2c_ref[0], y2c_ref[0]
    w3 = jnp.maximum(jnp.minimum(x2c, x2r) - jnp.maximum(x1c, x1r), 0.0) * 3.0
    h = jnp.maximum(jnp.minimum(y2c, y2r) - jnp.maximum(y1c, y1r), 0.0)
    inter3 = w3 * h
    arc = (x2c - x1c) * (y2c - y1c) + 5e-10
    arr = (x2r - x1r) * (y2r - y1r) + 5e-10
    s = jnp.where(inter3 > (arc + arr), 1.0, 0.0)
    sp = jnp.dot(s, _pack_matrix(), preferred_element_type=jnp.float32)
    sp_ref[...] = sp.astype(jnp.int32).reshape(1, KP, NW)


def _iou_pack(x1cm, y1cm, x2cm, y2cm):
    row = pl.BlockSpec((1, 1, KP), lambda c: (c, 0, 0))
    col = pl.BlockSpec((1, KP, 1), lambda c: (c, 0, 0))
    r3 = lambda a: a[:, None, :]
    c3 = lambda a: a[:, :, None]
    return pl.pallas_call(
        _iou_pack_body,
        grid=(NUM_FG,),
        in_specs=[row, row, row, row, col, col, col, col],
        out_specs=pl.BlockSpec((1, KP, NW), lambda c: (c, 0, 0)),
        out_shape=jax.ShapeDtypeStruct((NUM_FG, KP, NW), jnp.int32),
    )(r3(x1cm), r3(y1cm), r3(x2cm), r3(y2cm),
      c3(x1cm), c3(y1cm), c3(x2cm), c3(y2cm))


def _nms_topk_body(sp_ref, vals_ref, x1_ref, y1_ref, x2_ref, y2_ref,
                   x1o_ref, y1o_ref, x2o_ref, y2o_ref, so_ref, scr_ref):
    vals = vals_ref[...]
    validf = jnp.where(vals > SCORE_THRESH, 1.0, 0.0)
    keep = jnp.dot(validf, _pack_matrix(),
                   preferred_element_type=jnp.float32).astype(jnp.int32)

    lane = jax.lax.broadcasted_iota(jnp.int32, (1, NW), 1)
    for w in range(NW):
        later_words = jnp.where(lane > w, jnp.int32(-1), jnp.int32(0))
        for b in range(16):
            g = 16 * w + b
            srow = sp_ref[:, g, :]
            kb = (keep[:, w:w + 1] >> b) & 1
            if b == 15:
                fmask = later_words
            else:
                cur = ((0xFFFF << (b + 1)) & 0xFFFF)
                fmask = later_words | jnp.where(lane == w, jnp.int32(cur), 0)
            keep = keep & ~(srow & fmask & (-kb))

    bit16 = jax.lax.broadcasted_iota(jnp.int32, (1, 16), 1)
    for w in range(NW):
        bits = (keep[:, w:w + 1] >> bit16) & 1
        v = vals[:, 16 * w:16 * (w + 1)]
        scr_ref[:, 16 * w:16 * (w + 1)] = jnp.where(bits == 1, v, -1.0)

    ci = jax.lax.broadcasted_iota(jnp.int32, (NUM_FG, KP), 0)
    ri = jax.lax.broadcasted_iota(jnp.int32, (NUM_FG, KP), 1)
    flat = jnp.where(ri < K, ci * K + ri, BIG)
    x1v, y1v, x2v, y2v = x1_ref[...], y1_ref[...], x2_ref[...], y2_ref[...]

    def body(i, _):
        s = scr_ref[...]
        m = jnp.max(s)
        fr = jnp.where(s == m, flat, BIG)
        am = jnp.min(fr)
        sel = fr == am
        self32 = jnp.where(sel, 1.0, 0.0)
        x1o_ref[pl.ds(i, 1), :] = jnp.sum(self32 * x1v).reshape(1, 1)
        y1o_ref[pl.ds(i, 1), :] = jnp.sum(self32 * y1v).reshape(1, 1)
        x2o_ref[pl.ds(i, 1), :] = jnp.sum(self32 * x2v).reshape(1, 1)
        y2o_ref[pl.ds(i, 1), :] = jnp.sum(self32 * y2v).reshape(1, 1)
        so_ref[pl.ds(i, 1), :] = m.reshape(1, 1)
        scr_ref[...] = jnp.where(sel, -2.0, s)
        return 0

    jax.lax.fori_loop(0, DET_PER_IMG, body, 0)


def _nms_topk(s_pack, vals_cm, x1cm, y1cm, x2cm, y2cm):
    o = jax.ShapeDtypeStruct((DET_PER_IMG, 1), jnp.float32)
    return pl.pallas_call(
        _nms_topk_body,
        out_shape=(o, o, o, o, o),
        scratch_shapes=[pltpu.VMEM((NUM_FG, KP), jnp.float32)],
    )(s_pack, vals_cm, x1cm, y1cm, x2cm, y2cm)


def kernel(class_logits, box_regression, proposals):
    prob, x1, y1, x2, y2 = _softmax_decode(class_logits, box_regression, proposals)
    prob_c = prob[:, 1:].T
    vals, idx = jax.lax.top_k(prob_c, K)
    x1g = jnp.take_along_axis(x1[:, 1:].T, idx, axis=1)
    y1g = jnp.take_along_axis(y1[:, 1:].T, idx, axis=1)
    x2g = jnp.take_along_axis(x2[:, 1:].T, idx, axis=1)
    y2g = jnp.take_along_axis(y2[:, 1:].T, idx, axis=1)

    pad = ((0, 0), (0, KP - K))
    vals_cm = jnp.pad(vals, pad, constant_values=-1.0)
    x1cm = jnp.pad(x1g, pad)
    y1cm = jnp.pad(y1g, pad)
    x2cm = jnp.pad(x2g, pad)
    y2cm = jnp.pad(y2g, pad)

    s_pack = _iou_pack(x1cm, y1cm, x2cm, y2cm)
    x1o, y1o, x2o, y2o, so = _nms_topk(s_pack, vals_cm, x1cm, y1cm, x2cm, y2cm)
    return jnp.concatenate([x1o, y1o, x2o, y2o, so], axis=1)


# speedup vs baseline: 7.5751x; 1.6993x over previous
import functools
import math

import jax
import jax.numpy as jnp
from jax import lax
from jax.experimental import pallas as pl
from jax.experimental.pallas import tpu as pltpu
from jax.experimental.pallas import tpu_sc as plsc

N_PROP = 5000
NUM_CLASSES = 81
NUM_FG = 80
K = 500
KP = 512
NW = KP // 16
SCORE_THRESH = 0.05
DET_PER_IMG = 100
IMG_H = 800.0
IMG_W = 800.0
BBOX_XFORM_CLIP = math.log(1000.0 / 16.0)
BIG = 2**30


def _softmax_decode_body(logits_ref, dx_ref, dy_ref, dw_ref, dh_ref, props_ref,
                         prob_ref, x1_ref, y1_ref, x2_ref, y2_ref):
    logits = logits_ref[...]
    m = jnp.max(logits, axis=1, keepdims=True)
    e = jnp.exp(logits - m)
    prob_ref[...] = e / jnp.sum(e, axis=1, keepdims=True)

    p = props_ref[...]
    widths = p[:, 2:3] - p[:, 0:1]
    heights = p[:, 3:4] - p[:, 1:2]
    ctr_x = p[:, 0:1] + 0.5 * widths
    ctr_y = p[:, 1:2] + 0.5 * heights

    dx = dx_ref[...] * 0.1
    dy = dy_ref[...] * 0.1
    dw = jnp.minimum(dw_ref[...] * 0.2, BBOX_XFORM_CLIP)
    dh = jnp.minimum(dh_ref[...] * 0.2, BBOX_XFORM_CLIP)

    pcx = dx * widths + ctr_x
    pcy = dy * heights + ctr_y
    pw = jnp.exp(dw) * widths
    ph = jnp.exp(dh) * heights

    x1_ref[...] = jnp.clip(pcx - 0.5 * pw, 0.0, IMG_W)
    y1_ref[...] = jnp.clip(pcy - 0.5 * ph, 0.0, IMG_H)
    x2_ref[...] = jnp.clip(pcx + 0.5 * pw, 0.0, IMG_W)
    y2_ref[...] = jnp.clip(pcy + 0.5 * ph, 0.0, IMG_H)


def _softmax_decode(class_logits, box_regression, proposals):
    rel = box_regression.reshape(N_PROP, NUM_CLASSES, 4)
    shp = jax.ShapeDtypeStruct((N_PROP, NUM_CLASSES), jnp.float32)
    return pl.pallas_call(
        _softmax_decode_body,
        out_shape=(shp, shp, shp, shp, shp),
    )(class_logits, rel[..., 0], rel[..., 1], rel[..., 2], rel[..., 3], proposals)


NPAD = 5120
NCHUNK = NPAD // 16
BITS_THRESH = 0x3D4CCCCD
BITS_ONE = 0x3F800000


def _less(av, ai, bv, bi):
    return (av > bv) | ((av == bv) & (ai < bi))


def _make_sc_select():
    mesh = plsc.VectorSubcoreMesh(core_axis_name="c", subcore_axis_name="s",
                                  num_cores=2, num_subcores=16)
    o = jax.ShapeDtypeStruct((NUM_FG, KP), jnp.float32)

    @functools.partial(
        pl.kernel,
        out_type=(o, o, o, o, o),
        mesh=mesh,
        compiler_params=pltpu.CompilerParams(needs_layout_passes=False),
        scratch_types=[
            pltpu.VMEM((NPAD,), jnp.float32),
            pltpu.VMEM((NPAD,), jnp.float32),
            pltpu.VMEM((NPAD,), jnp.float32),
            pltpu.VMEM((NPAD,), jnp.float32),
            pltpu.VMEM((NPAD,), jnp.float32),
            pltpu.VMEM((NPAD,), jnp.float32),
            pltpu.VMEM((NPAD,), jnp.int32),
            pltpu.VMEM((KP,), jnp.float32),
            pltpu.VMEM((KP,), jnp.int32),
            pltpu.VMEM((KP,), jnp.float32),
            pltpu.VMEM((KP,), jnp.float32),
            pltpu.VMEM((KP,), jnp.float32),
            pltpu.VMEM((KP,), jnp.float32),
        ],
    )
    def sc_select(probs_hbm, x1_hbm, y1_hbm, x2_hbm, y2_hbm,
                  ov_hbm, ox1_hbm, oy1_hbm, ox2_hbm, oy2_hbm,
                  vals, bx1, by1, bx2, by2, cval, cidx, fval, fidx,
                  gx1, gy1, gx2, gy2):
        wid = lax.axis_index("s") * 2 + lax.axis_index("c")
        iota16 = lax.broadcasted_iota(jnp.int32, (16,), 0)
        ones16 = jnp.ones((16,), jnp.int32)
        zeros16 = jnp.zeros((16,), jnp.int32)
        thr16 = jnp.full((16,), SCORE_THRESH, jnp.float32)

        def do_class(c):
            pltpu.sync_copy(probs_hbm.at[c], vals)
            pltpu.sync_copy(x1_hbm.at[c], bx1)
            pltpu.sync_copy(y1_hbm.at[c], by1)
            pltpu.sync_copy(x2_hbm.at[c], bx2)
            pltpu.sync_copy(y2_hbm.at[c], by2)

            def p1(k, off):
                v = vals[pl.ds(k * 16, 16)]
                m = v > thr16
                mi = jnp.where(m, ones16, zeros16)
                cs = plsc.cumsum(mi)
                pos = off + cs - 1
                plsc.store_scatter(cval, [pos], v, mask=m)
                plsc.store_scatter(cidx, [pos], k * 16 + iota16, mask=m)
                return off + jnp.sum(mi)

            C = lax.fori_loop(0, NCHUNK, p1, jnp.int32(0))
            plsc.store_scatter(cval, [C + iota16],
                               jnp.full((16,), -1.0, jnp.float32))
            plsc.store_scatter(cidx, [C + iota16], zeros16)
            nch = (C + 15) // 16

            def bisect(_):
                def cnt_gt(tbits):
                    tv = plsc.bitcast(jnp.full((16,), tbits, jnp.int32),
                                      jnp.float32)

                    def cb(k, acc):
                        v = cval[pl.ds(k * 16, 16)]
                        return acc + jnp.where(v > tv, ones16, zeros16)

                    acc = lax.fori_loop(0, nch, cb, zeros16)
                    return jnp.sum(acc)

                def bb(_, lohi):
                    lo, hi = lohi
                    mid = (lo + hi) // 2
                    big = cnt_gt(mid) >= K
                    return (jnp.where(big, mid, lo), jnp.where(big, hi, mid))

                lo, hi = lax.fori_loop(0, 26, bb,
                                       (jnp.int32(BITS_THRESH),
                                        jnp.int32(BITS_ONE)))
                return hi

            vstar_bits = lax.cond(C > K, bisect,
                                  lambda _: jnp.int32(BITS_THRESH), 0)
            vstar = plsc.bitcast(jnp.full((16,), vstar_bits, jnp.int32),
                                 jnp.float32)

            def pf(k, _):
                fval[pl.ds(k * 16, 16)] = jnp.full((16,), -1.0, jnp.float32)
                fidx[pl.ds(k * 16, 16)] = zeros16
                return 0

            lax.fori_loop(0, KP // 16, pf, 0)

            def p2(k, off):
                v = cval[pl.ds(k * 16, 16)]
                ii = cidx[pl.ds(k * 16, 16)]
                m = v > vstar
                mi = jnp.where(m, ones16, zeros16)
                cs = plsc.cumsum(mi)
                pos = off + cs - 1
                mm = m & (pos < K)
                plsc.store_scatter(fval, [pos], v, mask=mm)
                plsc.store_scatter(fidx, [pos], ii, mask=mm)
                return off + jnp.sum(mi)

            G = lax.fori_loop(0, nch, p2, jnp.int32(0))

            @pl.when(C > K)
            def _():
                def p2b(k, off):
                    v = cval[pl.ds(k * 16, 16)]
                    ii = cidx[pl.ds(k * 16, 16)]
                    m = v == vstar
                    mi = jnp.where(m, ones16, zeros16)
                    cs = plsc.cumsum(mi)
                    pos = G + off + cs - 1
                    mm = m & (pos < K)
                    plsc.store_scatter(fval, [pos], v, mask=mm)
                    plsc.store_scatter(fidx, [pos], ii, mask=mm)
                    return off + jnp.sum(mi)

                lax.fori_loop(0, nch, p2b, jnp.int32(0))

            def cross_body(t, a):
                j16 = jnp.int32(1) << (a - 1 - t - 4)
                k = jnp.int32(1) << a
                for m in range(KP // 16):
                    @pl.when((m & j16) == 0)
                    def _():
                        pm = (m + j16) * 16
                        av = fval[pl.ds(m * 16, 16)]
                        ai = fidx[pl.ds(m * 16, 16)]
                        bv = fval[pl.ds(pm, 16)]
                        bi = fidx[pl.ds(pm, 16)]
                        asc = ((m * 16) & k) == 0
                        t1 = _less(av, ai, bv, bi)
                        cond = t1 == asc
                        fval[pl.ds(m * 16, 16)] = jnp.where(cond, av, bv)
                        fidx[pl.ds(m * 16, 16)] = jnp.where(cond, ai, bi)
                        fval[pl.ds(pm, 16)] = jnp.where(cond, bv, av)
                        fidx[pl.ds(pm, 16)] = jnp.where(cond, bi, ai)
                return a

            def intra_body(t, a):
                j = jnp.int32(8) >> t
                k = jnp.int32(1) << a
                perm = iota16 ^ j
                is_lower = (iota16 & j) == 0
                for m in range(KP // 16):
                    d_lane = ((iota16 + m * 16) & k) == 0
                    mv = fval[pl.ds(m * 16, 16)]
                    mi_ = fidx[pl.ds(m * 16, 16)]
                    pv = jnp.take_along_axis(mv, perm, axis=0)
                    pi = jnp.take_along_axis(mi_, perm, axis=0)
                    lmp = _less(mv, mi_, pv, pi)
                    lpm = _less(pv, pi, mv, mi_)
                    cond0 = jnp.where(is_lower, lmp, lpm)
                    cond = cond0 == d_lane
                    fval[pl.ds(m * 16, 16)] = jnp.where(cond, mv, pv)
                    fidx[pl.ds(m * 16, 16)] = jnp.where(cond, mi_, pi)
                return a

            def stage(a, _):
                lax.fori_loop(0, jnp.maximum(a - 4, 0), cross_body, a)
                lax.fori_loop(jnp.maximum(4 - a, 0), 4, intra_body, a)
                return 0

            lax.fori_loop(1, 10, stage, 0)

            def gb(k, _):
                ii = fidx[pl.ds(k * 16, 16)]
                gx1[pl.ds(k * 16, 16)] = plsc.load_gather(bx1, [ii])
                gy1[pl.ds(k * 16, 16)] = plsc.load_gather(by1, [ii])
                gx2[pl.ds(k * 16, 16)] = plsc.load_gather(bx2, [ii])
                gy2[pl.ds(k * 16, 16)] = plsc.load_gather(by2, [ii])
                return 0

            lax.fori_loop(0, KP // 16, gb, 0)

            pltpu.sync_copy(fval, ov_hbm.at[c])
            pltpu.sync_copy(gx1, ox1_hbm.at[c])
            pltpu.sync_copy(gy1, oy1_hbm.at[c])
            pltpu.sync_copy(gx2, ox2_hbm.at[c])
            pltpu.sync_copy(gy2, oy2_hbm.at[c])

        def rep_body(rep, _):
            c = wid + 32 * rep

            @pl.when(c < NUM_FG)
            def _():
                do_class(c)

            return 0

        lax.fori_loop(0, 3, rep_body, 0)

    return sc_select


def _pack_matrix():
    j16 = jax.lax.broadcasted_iota(jnp.int32, (KP, NW), 0)
    wl = jax.lax.broadcasted_iota(jnp.int32, (KP, NW), 1)
    pw = jnp.int32(1) << (j16 % 16)
    return jnp.where((j16 // 16) == wl, pw, 0).astype(jnp.float32)


def _iou_pack_body(x1r_ref, y1r_ref, x2r_ref, y2r_ref,
                   x1c_ref, y1c_ref, x2c_ref, y2c_ref, sp_ref):
    x1r, y1r, x2r, y2r = x1r_ref[0], y1r_ref[0], x2r_ref[0], y2r_ref[0]
    x1c, y1c, x2c, y2c = x1c_ref[0], y1c_ref[0], x2c_ref[0], y2c_ref[0]
    w3 = jnp.maximum(jnp.minimum(x2c, x2r) - jnp.maximum(x1c, x1r), 0.0) * 3.0
    h = jnp.maximum(jnp.minimum(y2c, y2r) - jnp.maximum(y1c, y1r), 0.0)
    inter3 = w3 * h
    arc = (x2c - x1c) * (y2c - y1c) + 5e-10
    arr = (x2r - x1r) * (y2r - y1r) + 5e-10
    s = jnp.where(inter3 > (arc + arr), 1.0, 0.0)
    sp = jnp.dot(s, _pack_matrix(), preferred_element_type=jnp.float32)
    sp_ref[...] = sp.astype(jnp.int32).reshape(1, KP, NW)


def _iou_pack(x1cm, y1cm, x2cm, y2cm):
    row = pl.BlockSpec((1, 1, KP), lambda c: (c, 0, 0))
    col = pl.BlockSpec((1, KP, 1), lambda c: (c, 0, 0))
    r3 = lambda a: a[:, None, :]
    c3 = lambda a: a[:, :, None]
    return pl.pallas_call(
        _iou_pack_body,
        grid=(NUM_FG,),
        in_specs=[row, row, row, row, col, col, col, col],
        out_specs=pl.BlockSpec((1, KP, NW), lambda c: (c, 0, 0)),
        out_shape=jax.ShapeDtypeStruct((NUM_FG, KP, NW), jnp.int32),
    )(r3(x1cm), r3(y1cm), r3(x2cm), r3(y2cm),
      c3(x1cm), c3(y1cm), c3(x2cm), c3(y2cm))


def _nms_topk_body(sp_ref, vals_ref, x1_ref, y1_ref, x2_ref, y2_ref,
                   x1o_ref, y1o_ref, x2o_ref, y2o_ref, so_ref, scr_ref):
    vals = vals_ref[...]
    validf = jnp.where(vals > SCORE_THRESH, 1.0, 0.0)
    keep = jnp.dot(validf, _pack_matrix(),
                   preferred_element_type=jnp.float32).astype(jnp.int32)

    lane = jax.lax.broadcasted_iota(jnp.int32, (1, NW), 1)
    for w in range(NW):
        later_words = jnp.where(lane > w, jnp.int32(-1), jnp.int32(0))
        for b in range(16):
            g = 16 * w + b
            srow = sp_ref[:, g, :]
            kb = (keep[:, w:w + 1] >> b) & 1
            if b == 15:
                fmask = later_words
            else:
                cur = ((0xFFFF << (b + 1)) & 0xFFFF)
                fmask = later_words | jnp.where(lane == w, jnp.int32(cur), 0)
            keep = keep & ~(srow & fmask & (-kb))

    bit16 = jax.lax.broadcasted_iota(jnp.int32, (1, 16), 1)
    for w in range(NW):
        bits = (keep[:, w:w + 1] >> bit16) & 1
        v = vals[:, 16 * w:16 * (w + 1)]
        scr_ref[:, 16 * w:16 * (w + 1)] = jnp.where(bits == 1, v, -1.0)

    ci = jax.lax.broadcasted_iota(jnp.int32, (NUM_FG, KP), 0)
    ri = jax.lax.broadcasted_iota(jnp.int32, (NUM_FG, KP), 1)
    flat = jnp.where(ri < K, ci * K + ri, BIG)
    x1v, y1v, x2v, y2v = x1_ref[...], y1_ref[...], x2_ref[...], y2_ref[...]

    def body(i, _):
        s = scr_ref[...]
        m = jnp.max(s)
        fr = jnp.where(s == m, flat, BIG)
        am = jnp.min(fr)
        sel = fr == am
        self32 = jnp.where(sel, 1.0, 0.0)
        x1o_ref[pl.ds(i, 1), :] = jnp.sum(self32 * x1v).reshape(1, 1)
        y1o_ref[pl.ds(i, 1), :] = jnp.sum(self32 * y1v).reshape(1, 1)
        x2o_ref[pl.ds(i, 1), :] = jnp.sum(self32 * x2v).reshape(1, 1)
        y2o_ref[pl.ds(i, 1), :] = jnp.sum(self32 * y2v).reshape(1, 1)
        so_ref[pl.ds(i, 1), :] = m.reshape(1, 1)
        scr_ref[...] = jnp.where(sel, -2.0, s)
        return 0

    jax.lax.fori_loop(0, DET_PER_IMG, body, 0)


def _nms_topk(s_pack, vals_cm, x1cm, y1cm, x2cm, y2cm):
    o = jax.ShapeDtypeStruct((DET_PER_IMG, 1), jnp.float32)
    return pl.pallas_call(
        _nms_topk_body,
        out_shape=(o, o, o, o, o),
        scratch_shapes=[pltpu.VMEM((NUM_FG, KP), jnp.float32)],
    )(s_pack, vals_cm, x1cm, y1cm, x2cm, y2cm)


def kernel(class_logits, box_regression, proposals):
    prob, x1, y1, x2, y2 = _softmax_decode(class_logits, box_regression, proposals)
    padn = ((0, 0), (0, NPAD - N_PROP))
    probs_cm = jnp.pad(prob[:, 1:].T, padn, constant_values=-1.0)
    x1p = jnp.pad(x1[:, 1:].T, padn)
    y1p = jnp.pad(y1[:, 1:].T, padn)
    x2p = jnp.pad(x2[:, 1:].T, padn)
    y2p = jnp.pad(y2[:, 1:].T, padn)

    vals_cm, x1cm, y1cm, x2cm, y2cm = _make_sc_select()(
        probs_cm, x1p, y1p, x2p, y2p)

    s_pack = _iou_pack(x1cm, y1cm, x2cm, y2cm)
    x1o, y1o, x2o, y2o, so = _nms_topk(s_pack, vals_cm, x1cm, y1cm, x2cm, y2cm)
    return jnp.concatenate([x1o, y1o, x2o, y2o, so], axis=1)


# speedup vs baseline: 8.2053x; 1.0832x over previous
import functools
import math

import jax
import jax.numpy as jnp
from jax import lax
from jax.experimental import pallas as pl
from jax.experimental.pallas import tpu as pltpu
from jax.experimental.pallas import tpu_sc as plsc

N_PROP = 5000
NUM_CLASSES = 81
NUM_FG = 80
K = 500
KP = 512
NW = KP // 16
SCORE_THRESH = 0.05
DET_PER_IMG = 100
IMG_H = 800.0
IMG_W = 800.0
BBOX_XFORM_CLIP = math.log(1000.0 / 16.0)
BIG = 2**30


def _sel_matrix(k):
    j = jax.lax.broadcasted_iota(jnp.int32, (4 * NUM_CLASSES, NUM_FG), 0)
    cc = jax.lax.broadcasted_iota(jnp.int32, (4 * NUM_CLASSES, NUM_FG), 1)
    return jnp.where(j == 4 * (cc + 1) + k, 1.0, 0.0)


RB = 1280


def _t_cm(a, padval, rmask):
    a = jnp.where(rmask, a, padval)
    a128 = jnp.concatenate(
        [a, jnp.full((RB, 128 - NUM_FG), padval, jnp.float32)], axis=1)
    return jnp.transpose(a128)[:NUM_FG, :]


def _softmax_decode_body(logits_ref, br_ref, props_ref,
                         prob_ref, x1_ref, y1_ref, x2_ref, y2_ref):
    pid = pl.program_id(0)
    rid = pid * RB + jax.lax.broadcasted_iota(jnp.int32, (RB, 1), 0)
    rmask = rid < N_PROP
    logits = logits_ref[...]
    m = jnp.max(logits, axis=1, keepdims=True)
    e = jnp.exp(logits - m)
    prob = e / jnp.sum(e, axis=1, keepdims=True)

    p = props_ref[...]
    widths = p[:, 2:3] - p[:, 0:1]
    heights = p[:, 3:4] - p[:, 1:2]
    ctr_x = p[:, 0:1] + 0.5 * widths
    ctr_y = p[:, 1:2] + 0.5 * heights

    br = br_ref[...]
    hp = jax.lax.Precision.HIGHEST
    dx = jnp.dot(br, _sel_matrix(0), precision=hp) * 0.1
    dy = jnp.dot(br, _sel_matrix(1), precision=hp) * 0.1
    dw = jnp.minimum(jnp.dot(br, _sel_matrix(2), precision=hp) * 0.2,
                     BBOX_XFORM_CLIP)
    dh = jnp.minimum(jnp.dot(br, _sel_matrix(3), precision=hp) * 0.2,
                     BBOX_XFORM_CLIP)

    pcx = dx * widths + ctr_x
    pcy = dy * heights + ctr_y
    pw = jnp.exp(dw) * widths
    ph = jnp.exp(dh) * heights

    prob_ref[...] = _t_cm(prob[:, 1:], -1.0, rmask)
    x1_ref[...] = _t_cm(jnp.clip(pcx - 0.5 * pw, 0.0, IMG_W), 0.0, rmask)
    y1_ref[...] = _t_cm(jnp.clip(pcy - 0.5 * ph, 0.0, IMG_H), 0.0, rmask)
    x2_ref[...] = _t_cm(jnp.clip(pcx + 0.5 * pw, 0.0, IMG_W), 0.0, rmask)
    y2_ref[...] = _t_cm(jnp.clip(pcy + 0.5 * ph, 0.0, IMG_H), 0.0, rmask)


def _softmax_decode(class_logits, box_regression, proposals):
    shp = jax.ShapeDtypeStruct((NUM_FG, NPAD), jnp.float32)
    ospec = pl.BlockSpec((NUM_FG, RB), lambda i: (0, i))
    return pl.pallas_call(
        _softmax_decode_body,
        grid=(NPAD // RB,),
        in_specs=[
            pl.BlockSpec((RB, NUM_CLASSES), lambda i: (i, 0)),
            pl.BlockSpec((RB, 4 * NUM_CLASSES), lambda i: (i, 0)),
            pl.BlockSpec((RB, 4), lambda i: (i, 0)),
        ],
        out_specs=(ospec, ospec, ospec, ospec, ospec),
        out_shape=(shp, shp, shp, shp, shp),
    )(class_logits, box_regression, proposals)


NPAD = 5120
NCHUNK = NPAD // 16
BITS_THRESH = 0x3D4CCCCD
BITS_ONE = 0x3F800000


def _less(av, ai, bv, bi):
    return (av > bv) | ((av == bv) & (ai < bi))


def _make_sc_select():
    mesh = plsc.VectorSubcoreMesh(core_axis_name="c", subcore_axis_name="s",
                                  num_cores=2, num_subcores=16)
    o = jax.ShapeDtypeStruct((NUM_FG, KP), jnp.float32)

    @functools.partial(
        pl.kernel,
        out_type=(o, o, o, o, o),
        mesh=mesh,
        compiler_params=pltpu.CompilerParams(needs_layout_passes=False),
        scratch_types=[
            pltpu.VMEM((NPAD,), jnp.float32),
            pltpu.VMEM((NPAD,), jnp.float32),
            pltpu.VMEM((NPAD,), jnp.float32),
            pltpu.VMEM((NPAD,), jnp.float32),
            pltpu.VMEM((NPAD,), jnp.float32),
            pltpu.VMEM((NPAD,), jnp.float32),
            pltpu.VMEM((NPAD,), jnp.int32),
            pltpu.VMEM((KP,), jnp.float32),
            pltpu.VMEM((KP,), jnp.int32),
            pltpu.VMEM((KP,), jnp.float32),
            pltpu.VMEM((KP,), jnp.float32),
            pltpu.VMEM((KP,), jnp.float32),
            pltpu.VMEM((KP,), jnp.float32),
        ],
    )
    def sc_select(probs_hbm, x1_hbm, y1_hbm, x2_hbm, y2_hbm,
                  ov_hbm, ox1_hbm, oy1_hbm, ox2_hbm, oy2_hbm,
                  vals, bx1, by1, bx2, by2, cval, cidx, fval, fidx,
                  gx1, gy1, gx2, gy2):
        wid = lax.axis_index("s") * 2 + lax.axis_index("c")
        iota16 = lax.broadcasted_iota(jnp.int32, (16,), 0)
        ones16 = jnp.ones((16,), jnp.int32)
        zeros16 = jnp.zeros((16,), jnp.int32)
        thr16 = jnp.full((16,), SCORE_THRESH, jnp.float32)

        def do_class(c):
            pltpu.sync_copy(probs_hbm.at[c], vals)
            pltpu.sync_copy(x1_hbm.at[c], bx1)
            pltpu.sync_copy(y1_hbm.at[c], by1)
            pltpu.sync_copy(x2_hbm.at[c], bx2)
            pltpu.sync_copy(y2_hbm.at[c], by2)

            def p1(k, off):
                v = vals[pl.ds(k * 16, 16)]
                m = v > thr16
                mi = jnp.where(m, ones16, zeros16)
                cs = plsc.cumsum(mi)
                pos = off + cs - 1
                plsc.store_scatter(cval, [pos], v, mask=m)
                plsc.store_scatter(cidx, [pos], k * 16 + iota16, mask=m)
                return off + jnp.sum(mi)

            C = lax.fori_loop(0, NCHUNK, p1, jnp.int32(0))
            plsc.store_scatter(cval, [C + iota16],
                               jnp.full((16,), -1.0, jnp.float32))
            plsc.store_scatter(cidx, [C + iota16], zeros16)
            nch = (C + 15) // 16

            def bisect(_):
                def cnt_gt(tbits):
                    tv = plsc.bitcast(jnp.full((16,), tbits, jnp.int32),
                                      jnp.float32)

                    def cb(k, acc):
                        v = cval[pl.ds(k * 16, 16)]
                        return acc + jnp.where(v > tv, ones16, zeros16)

                    acc = lax.fori_loop(0, nch, cb, zeros16)
                    return jnp.sum(acc)

                def bb(_, lohi):
                    lo, hi = lohi
                    mid = (lo + hi) // 2
                    big = cnt_gt(mid) >= K
                    return (jnp.where(big, mid, lo), jnp.where(big, hi, mid))

                lo, hi = lax.fori_loop(0, 26, bb,
                                       (jnp.int32(BITS_THRESH),
                                        jnp.int32(BITS_ONE)))
                return hi

            vstar_bits = lax.cond(C > K, bisect,
                                  lambda _: jnp.int32(BITS_THRESH), 0)
            vstar = plsc.bitcast(jnp.full((16,), vstar_bits, jnp.int32),
                                 jnp.float32)

            def pf(k, _):
                fval[pl.ds(k * 16, 16)] = jnp.full((16,), -1.0, jnp.float32)
                fidx[pl.ds(k * 16, 16)] = zeros16
                return 0

            lax.fori_loop(0, KP // 16, pf, 0)

            def p2(k, off):
                v = cval[pl.ds(k * 16, 16)]
                ii = cidx[pl.ds(k * 16, 16)]
                m = v > vstar
                mi = jnp.where(m, ones16, zeros16)
                cs = plsc.cumsum(mi)
                pos = off + cs - 1
                mm = m & (pos < K)
                plsc.store_scatter(fval, [pos], v, mask=mm)
                plsc.store_scatter(fidx, [pos], ii, mask=mm)
                return off + jnp.sum(mi)

            G = lax.fori_loop(0, nch, p2, jnp.int32(0))

            @pl.when(C > K)
            def _():
                def p2b(k, off):
                    v = cval[pl.ds(k * 16, 16)]
                    ii = cidx[pl.ds(k * 16, 16)]
                    m = v == vstar
                    mi = jnp.where(m, ones16, zeros16)
                    cs = plsc.cumsum(mi)
                    pos = G + off + cs - 1
                    mm = m & (pos < K)
                    plsc.store_scatter(fval, [pos], v, mask=mm)
                    plsc.store_scatter(fidx, [pos], ii, mask=mm)
                    return off + jnp.sum(mi)

                lax.fori_loop(0, nch, p2b, jnp.int32(0))

            def cross_body(t, a):
                j16 = jnp.int32(1) << (a - 1 - t - 4)
                k = jnp.int32(1) << a
                for m in range(KP // 16):
                    @pl.when((m & j16) == 0)
                    def _():
                        pm = (m + j16) * 16
                        av = fval[pl.ds(m * 16, 16)]
                        ai = fidx[pl.ds(m * 16, 16)]
                        bv = fval[pl.ds(pm, 16)]
                        bi = fidx[pl.ds(pm, 16)]
                        asc = ((m * 16) & k) == 0
                        t1 = _less(av, ai, bv, bi)
                        cond = t1 == asc
                        fval[pl.ds(m * 16, 16)] = jnp.where(cond, av, bv)
                        fidx[pl.ds(m * 16, 16)] = jnp.where(cond, ai, bi)
                        fval[pl.ds(pm, 16)] = jnp.where(cond, bv, av)
                        fidx[pl.ds(pm, 16)] = jnp.where(cond, bi, ai)
                return a

            def intra_body(t, a):
                j = jnp.int32(8) >> t
                k = jnp.int32(1) << a
                perm = iota16 ^ j
                is_lower = (iota16 & j) == 0
                for m in range(KP // 16):
                    d_lane = ((iota16 + m * 16) & k) == 0
                    mv = fval[pl.ds(m * 16, 16)]
                    mi_ = fidx[pl.ds(m * 16, 16)]
                    pv = jnp.take_along_axis(mv, perm, axis=0)
                    pi = jnp.take_along_axis(mi_, perm, axis=0)
                    lmp = _less(mv, mi_, pv, pi)
                    lpm = _less(pv, pi, mv, mi_)
                    cond0 = jnp.where(is_lower, lmp, lpm)
                    cond = cond0 == d_lane
                    fval[pl.ds(m * 16, 16)] = jnp.where(cond, mv, pv)
                    fidx[pl.ds(m * 16, 16)] = jnp.where(cond, mi_, pi)
                return a

            def stage(a, _):
                lax.fori_loop(0, jnp.maximum(a - 4, 0), cross_body, a)
                lax.fori_loop(jnp.maximum(4 - a, 0), 4, intra_body, a)
                return 0

            lax.fori_loop(1, 10, stage, 0)

            def gb(k, _):
                ii = fidx[pl.ds(k * 16, 16)]
                gx1[pl.ds(k * 16, 16)] = plsc.load_gather(bx1, [ii])
                gy1[pl.ds(k * 16, 16)] = plsc.load_gather(by1, [ii])
                gx2[pl.ds(k * 16, 16)] = plsc.load_gather(bx2, [ii])
                gy2[pl.ds(k * 16, 16)] = plsc.load_gather(by2, [ii])
                return 0

            lax.fori_loop(0, KP // 16, gb, 0)

            pltpu.sync_copy(fval, ov_hbm.at[c])
            pltpu.sync_copy(gx1, ox1_hbm.at[c])
            pltpu.sync_copy(gy1, oy1_hbm.at[c])
            pltpu.sync_copy(gx2, ox2_hbm.at[c])
            pltpu.sync_copy(gy2, oy2_hbm.at[c])

        def rep_body(rep, _):
            c = wid + 32 * rep

            @pl.when(c < NUM_FG)
            def _():
                do_class(c)

            return 0

        lax.fori_loop(0, 3, rep_body, 0)

    return sc_select


def _pack_matrix():
    j16 = jax.lax.broadcasted_iota(jnp.int32, (KP, NW), 0)
    wl = jax.lax.broadcasted_iota(jnp.int32, (KP, NW), 1)
    pw = jnp.int32(1) << (j16 % 16)
    return jnp.where((j16 // 16) == wl, pw, 0).astype(jnp.float32)


def _iou_pack_body(x1r_ref, y1r_ref, x2r_ref, y2r_ref,
                   x1c_ref, y1c_ref, x2c_ref, y2c_ref, sp_ref):
    x1r, y1r, x2r, y2r = x1r_ref[0], y1r_ref[0], x2r_ref[0], y2r_ref[0]
    x1c, y1c, x2c, y2c = x1c_ref[0], y1c_ref[0], x2c_ref[0], y2c_ref[0]
    w3 = jnp.maximum(jnp.minimum(x2c, x2r) - jnp.maximum(x1c, x1r), 0.0) * 3.0
    h = jnp.maximum(jnp.minimum(y2c, y2r) - jnp.maximum(y1c, y1r), 0.0)
    inter3 = w3 * h
    arc = (x2c - x1c) * (y2c - y1c) + 5e-10
    arr = (x2r - x1r) * (y2r - y1r) + 5e-10
    s = jnp.where(inter3 > (arc + arr), 1.0, 0.0)
    sp = jnp.dot(s, _pack_matrix(), preferred_element_type=jnp.float32)
    sp_ref[...] = sp.astype(jnp.int32).reshape(1, KP, NW)


def _iou_pack(x1cm, y1cm, x2cm, y2cm):
    row = pl.BlockSpec((1, 1, KP), lambda c: (c, 0, 0))
    col = pl.BlockSpec((1, KP, 1), lambda c: (c, 0, 0))
    r3 = lambda a: a[:, None, :]
    c3 = lambda a: a[:, :, None]
    return pl.pallas_call(
        _iou_pack_body,
        grid=(NUM_FG,),
        in_specs=[row, row, row, row, col, col, col, col],
        out_specs=pl.BlockSpec((1, KP, NW), lambda c: (c, 0, 0)),
        out_shape=jax.ShapeDtypeStruct((NUM_FG, KP, NW), jnp.int32),
    )(r3(x1cm), r3(y1cm), r3(x2cm), r3(y2cm),
      c3(x1cm), c3(y1cm), c3(x2cm), c3(y2cm))


def _nms_topk_body(sp_ref, vals_ref, x1_ref, y1_ref, x2_ref, y2_ref,
                   x1o_ref, y1o_ref, x2o_ref, y2o_ref, so_ref, scr_ref):
    vals = vals_ref[...]
    validf = jnp.where(vals > SCORE_THRESH, 1.0, 0.0)
    keep = jnp.dot(validf, _pack_matrix(),
                   preferred_element_type=jnp.float32).astype(jnp.int32)

    lane = jax.lax.broadcasted_iota(jnp.int32, (1, NW), 1)
    for w in range(NW):
        later_words = jnp.where(lane > w, jnp.int32(-1), jnp.int32(0))
        for b in range(16):
            g = 16 * w + b
            srow = sp_ref[:, g, :]
            kb = (keep[:, w:w + 1] >> b) & 1
            if b == 15:
                fmask = later_words
            else:
                cur = ((0xFFFF << (b + 1)) & 0xFFFF)
                fmask = later_words | jnp.where(lane == w, jnp.int32(cur), 0)
            keep = keep & ~(srow & fmask & (-kb))

    bit16 = jax.lax.broadcasted_iota(jnp.int32, (1, 16), 1)
    for w in range(NW):
        bits = (keep[:, w:w + 1] >> bit16) & 1
        v = vals[:, 16 * w:16 * (w + 1)]
        scr_ref[:, 16 * w:16 * (w + 1)] = jnp.where(bits == 1, v, -1.0)

    ci = jax.lax.broadcasted_iota(jnp.int32, (NUM_FG, KP), 0)
    ri = jax.lax.broadcasted_iota(jnp.int32, (NUM_FG, KP), 1)
    flat = jnp.where(ri < K, ci * K + ri, BIG)
    x1v, y1v, x2v, y2v = x1_ref[...], y1_ref[...], x2_ref[...], y2_ref[...]

    def body(i, _):
        s = scr_ref[...]
        m = jnp.max(s)
        fr = jnp.where(s == m, flat, BIG)
        am = jnp.min(fr)
        sel = fr == am
        self32 = jnp.where(sel, 1.0, 0.0)
        x1o_ref[pl.ds(i, 1), :] = jnp.sum(self32 * x1v).reshape(1, 1)
        y1o_ref[pl.ds(i, 1), :] = jnp.sum(self32 * y1v).reshape(1, 1)
        x2o_ref[pl.ds(i, 1), :] = jnp.sum(self32 * x2v).reshape(1, 1)
        y2o_ref[pl.ds(i, 1), :] = jnp.sum(self32 * y2v).reshape(1, 1)
        so_ref[pl.ds(i, 1), :] = m.reshape(1, 1)
        scr_ref[...] = jnp.where(sel, -2.0, s)
        return 0

    jax.lax.fori_loop(0, DET_PER_IMG, body, 0)


def _nms_topk(s_pack, vals_cm, x1cm, y1cm, x2cm, y2cm):
    o = jax.ShapeDtypeStruct((DET_PER_IMG, 1), jnp.float32)
    return pl.pallas_call(
        _nms_topk_body,
        out_shape=(o, o, o, o, o),
        scratch_shapes=[pltpu.VMEM((NUM_FG, KP), jnp.float32)],
    )(s_pack, vals_cm, x1cm, y1cm, x2cm, y2cm)


def kernel(class_logits, box_regression, proposals):
    probs_cm, x1p, y1p, x2p, y2p = _softmax_decode(
        class_logits, box_regression, proposals)

    vals_cm, x1cm, y1cm, x2cm, y2cm = _make_sc_select()(
        probs_cm, x1p, y1p, x2p, y2p)

    s_pack = _iou_pack(x1cm, y1cm, x2cm, y2cm)
    x1o, y1o, x2o, y2o, so = _nms_topk(s_pack, vals_cm, x1cm, y1cm, x2cm, y2cm)
    return jnp.concatenate([x1o, y1o, x2o, y2o, so], axis=1)


# speedup vs baseline: 8.5721x; 1.0447x over previous
import functools
import math

import jax
import jax.numpy as jnp
from jax import lax
from jax.experimental import pallas as pl
from jax.experimental.pallas import tpu as pltpu
from jax.experimental.pallas import tpu_sc as plsc

N_PROP = 5000
NUM_CLASSES = 81
NUM_FG = 80
K = 500
KP = 512
NW = KP // 16
SCORE_THRESH = 0.05
DET_PER_IMG = 100
IMG_H = 800.0
IMG_W = 800.0
BBOX_XFORM_CLIP = math.log(1000.0 / 16.0)
BIG = 2**30


def _sel_matrix(k):
    j = jax.lax.broadcasted_iota(jnp.int32, (4 * NUM_CLASSES, NUM_FG), 0)
    cc = jax.lax.broadcasted_iota(jnp.int32, (4 * NUM_CLASSES, NUM_FG), 1)
    return jnp.where(j == 4 * (cc + 1) + k, 1.0, 0.0)


RB = 1280


def _t_cm(a, padval, rmask):
    a = jnp.where(rmask, a, padval)
    a128 = jnp.concatenate(
        [a, jnp.full((RB, 128 - NUM_FG), padval, jnp.float32)], axis=1)
    return jnp.transpose(a128)[:NUM_FG, :]


def _softmax_decode_body(logits_ref, br_ref, props_ref,
                         prob_ref, x1_ref, y1_ref, x2_ref, y2_ref):
    pid = pl.program_id(0)
    rid = pid * RB + jax.lax.broadcasted_iota(jnp.int32, (RB, 1), 0)
    rmask = rid < N_PROP
    logits = logits_ref[...]
    m = jnp.max(logits, axis=1, keepdims=True)
    e = jnp.exp(logits - m)
    prob = e / jnp.sum(e, axis=1, keepdims=True)

    p = props_ref[...]
    widths = p[:, 2:3] - p[:, 0:1]
    heights = p[:, 3:4] - p[:, 1:2]
    ctr_x = p[:, 0:1] + 0.5 * widths
    ctr_y = p[:, 1:2] + 0.5 * heights

    br = br_ref[...]
    hp = jax.lax.Precision.HIGHEST
    dx = jnp.dot(br, _sel_matrix(0), precision=hp) * 0.1
    dy = jnp.dot(br, _sel_matrix(1), precision=hp) * 0.1
    dw = jnp.minimum(jnp.dot(br, _sel_matrix(2), precision=hp) * 0.2,
                     BBOX_XFORM_CLIP)
    dh = jnp.minimum(jnp.dot(br, _sel_matrix(3), precision=hp) * 0.2,
                     BBOX_XFORM_CLIP)

    pcx = dx * widths + ctr_x
    pcy = dy * heights + ctr_y
    pw = jnp.exp(dw) * widths
    ph = jnp.exp(dh) * heights

    prob_ref[...] = _t_cm(prob[:, 1:], -1.0, rmask)
    x1_ref[...] = _t_cm(jnp.clip(pcx - 0.5 * pw, 0.0, IMG_W), 0.0, rmask)
    y1_ref[...] = _t_cm(jnp.clip(pcy - 0.5 * ph, 0.0, IMG_H), 0.0, rmask)
    x2_ref[...] = _t_cm(jnp.clip(pcx + 0.5 * pw, 0.0, IMG_W), 0.0, rmask)
    y2_ref[...] = _t_cm(jnp.clip(pcy + 0.5 * ph, 0.0, IMG_H), 0.0, rmask)


def _softmax_decode(class_logits, box_regression, proposals):
    shp = jax.ShapeDtypeStruct((NUM_FG, NPAD), jnp.float32)
    ospec = pl.BlockSpec((NUM_FG, RB), lambda i: (0, i))
    return pl.pallas_call(
        _softmax_decode_body,
        grid=(NPAD // RB,),
        in_specs=[
            pl.BlockSpec((RB, NUM_CLASSES), lambda i: (i, 0)),
            pl.BlockSpec((RB, 4 * NUM_CLASSES), lambda i: (i, 0)),
            pl.BlockSpec((RB, 4), lambda i: (i, 0)),
        ],
        out_specs=(ospec, ospec, ospec, ospec, ospec),
        out_shape=(shp, shp, shp, shp, shp),
    )(class_logits, box_regression, proposals)


NPAD = 5120
NCHUNK = NPAD // 16
BITS_THRESH = 0x3D4CCCCD
BITS_ONE = 0x3F800000


def _less(av, ai, bv, bi):
    return (av > bv) | ((av == bv) & (ai < bi))


def _make_sc_select():
    mesh = plsc.VectorSubcoreMesh(core_axis_name="c", subcore_axis_name="s",
                                  num_cores=2, num_subcores=16)
    o = jax.ShapeDtypeStruct((NUM_FG, KP), jnp.float32)

    @functools.partial(
        pl.kernel,
        out_type=(o, o, o, o, o),
        mesh=mesh,
        compiler_params=pltpu.CompilerParams(needs_layout_passes=False),
        scratch_types=[
            pltpu.VMEM((NPAD,), jnp.float32),
            pltpu.VMEM((NPAD,), jnp.float32),
            pltpu.VMEM((NPAD,), jnp.float32),
            pltpu.VMEM((NPAD,), jnp.float32),
            pltpu.VMEM((NPAD,), jnp.float32),
            pltpu.VMEM((NPAD,), jnp.float32),
            pltpu.VMEM((NPAD,), jnp.int32),
            pltpu.VMEM((KP,), jnp.float32),
            pltpu.VMEM((KP,), jnp.int32),
            pltpu.VMEM((KP,), jnp.float32),
            pltpu.VMEM((KP,), jnp.float32),
            pltpu.VMEM((KP,), jnp.float32),
            pltpu.VMEM((KP,), jnp.float32),
        ],
    )
    def sc_select(probs_hbm, x1_hbm, y1_hbm, x2_hbm, y2_hbm,
                  ov_hbm, ox1_hbm, oy1_hbm, ox2_hbm, oy2_hbm,
                  vals, bx1, by1, bx2, by2, cval, cidx, fval, fidx,
                  gx1, gy1, gx2, gy2):
        wid = lax.axis_index("s") * 2 + lax.axis_index("c")
        iota16 = lax.broadcasted_iota(jnp.int32, (16,), 0)
        ones16 = jnp.ones((16,), jnp.int32)
        zeros16 = jnp.zeros((16,), jnp.int32)
        thr16 = jnp.full((16,), SCORE_THRESH, jnp.float32)

        def do_class(c):
            pltpu.sync_copy(probs_hbm.at[c], vals)
            pltpu.sync_copy(x1_hbm.at[c], bx1)
            pltpu.sync_copy(y1_hbm.at[c], by1)
            pltpu.sync_copy(x2_hbm.at[c], bx2)
            pltpu.sync_copy(y2_hbm.at[c], by2)

            def p1(k, off):
                v = vals[pl.ds(k * 16, 16)]
                m = v > thr16
                mi = jnp.where(m, ones16, zeros16)
                cs = plsc.cumsum(mi)
                pos = off + cs - 1
                plsc.store_scatter(cval, [pos], v, mask=m)
                plsc.store_scatter(cidx, [pos], k * 16 + iota16, mask=m)
                return off + jnp.sum(mi)

            C = lax.fori_loop(0, NCHUNK, p1, jnp.int32(0))
            plsc.store_scatter(cval, [C + iota16],
                               jnp.full((16,), -1.0, jnp.float32))
            plsc.store_scatter(cidx, [C + iota16], zeros16)
            nch = (C + 15) // 16

            def bisect(_):
                def cnt_gt(tbits):
                    tv = plsc.bitcast(jnp.full((16,), tbits, jnp.int32),
                                      jnp.float32)

                    def cb(k, acc):
                        v = cval[pl.ds(k * 16, 16)]
                        return acc + jnp.where(v > tv, ones16, zeros16)

                    acc = lax.fori_loop(0, nch, cb, zeros16)
                    return jnp.sum(acc)

                def bb(_, lohi):
                    lo, hi = lohi
                    mid = (lo + hi) // 2
                    big = cnt_gt(mid) >= K
                    return (jnp.where(big, mid, lo), jnp.where(big, hi, mid))

                lo, hi = lax.fori_loop(0, 26, bb,
                                       (jnp.int32(BITS_THRESH),
                                        jnp.int32(BITS_ONE)))
                return hi

            vstar_bits = lax.cond(C > K, bisect,
                                  lambda _: jnp.int32(BITS_THRESH), 0)
            vstar = plsc.bitcast(jnp.full((16,), vstar_bits, jnp.int32),
                                 jnp.float32)

            def pf(k, _):
                fval[pl.ds(k * 16, 16)] = jnp.full((16,), -1.0, jnp.float32)
                fidx[pl.ds(k * 16, 16)] = zeros16
                return 0

            lax.fori_loop(0, KP // 16, pf, 0)

            def p2(k, off):
                v = cval[pl.ds(k * 16, 16)]
                ii = cidx[pl.ds(k * 16, 16)]
                m = v > vstar
                mi = jnp.where(m, ones16, zeros16)
                cs = plsc.cumsum(mi)
                pos = off + cs - 1
                mm = m & (pos < K)
                plsc.store_scatter(fval, [pos], v, mask=mm)
                plsc.store_scatter(fidx, [pos], ii, mask=mm)
                return off + jnp.sum(mi)

            G = lax.fori_loop(0, nch, p2, jnp.int32(0))

            @pl.when(C > K)
            def _():
                def p2b(k, off):
                    v = cval[pl.ds(k * 16, 16)]
                    ii = cidx[pl.ds(k * 16, 16)]
                    m = v == vstar
                    mi = jnp.where(m, ones16, zeros16)
                    cs = plsc.cumsum(mi)
                    pos = G + off + cs - 1
                    mm = m & (pos < K)
                    plsc.store_scatter(fval, [pos], v, mask=mm)
                    plsc.store_scatter(fidx, [pos], ii, mask=mm)
                    return off + jnp.sum(mi)

                lax.fori_loop(0, nch, p2b, jnp.int32(0))

            def cross_body(t, a):
                j16 = jnp.int32(1) << (a - 1 - t - 4)
                k = jnp.int32(1) << a
                for m in range(KP // 16):
                    @pl.when((m & j16) == 0)
                    def _():
                        pm = (m + j16) * 16
                        av = fval[pl.ds(m * 16, 16)]
                        ai = fidx[pl.ds(m * 16, 16)]
                        bv = fval[pl.ds(pm, 16)]
                        bi = fidx[pl.ds(pm, 16)]
                        asc = ((m * 16) & k) == 0
                        t1 = _less(av, ai, bv, bi)
                        cond = t1 == asc
                        fval[pl.ds(m * 16, 16)] = jnp.where(cond, av, bv)
                        fidx[pl.ds(m * 16, 16)] = jnp.where(cond, ai, bi)
                        fval[pl.ds(pm, 16)] = jnp.where(cond, bv, av)
                        fidx[pl.ds(pm, 16)] = jnp.where(cond, bi, ai)
                return a

            def intra_body(t, a):
                j = jnp.int32(8) >> t
                k = jnp.int32(1) << a
                perm = iota16 ^ j
                is_lower = (iota16 & j) == 0
                for m in range(KP // 16):
                    d_lane = ((iota16 + m * 16) & k) == 0
                    mv = fval[pl.ds(m * 16, 16)]
                    mi_ = fidx[pl.ds(m * 16, 16)]
                    pv = jnp.take_along_axis(mv, perm, axis=0)
                    pi = jnp.take_along_axis(mi_, perm, axis=0)
                    lmp = _less(mv, mi_, pv, pi)
                    lpm = _less(pv, pi, mv, mi_)
                    cond0 = jnp.where(is_lower, lmp, lpm)
                    cond = cond0 == d_lane
                    fval[pl.ds(m * 16, 16)] = jnp.where(cond, mv, pv)
                    fidx[pl.ds(m * 16, 16)] = jnp.where(cond, mi_, pi)
                return a

            def stage(a, _):
                lax.fori_loop(0, jnp.maximum(a - 4, 0), cross_body, a)
                lax.fori_loop(jnp.maximum(4 - a, 0), 4, intra_body, a)
                return 0

            lax.fori_loop(1, 10, stage, 0)

            def gb(k, _):
                ii = fidx[pl.ds(k * 16, 16)]
                gx1[pl.ds(k * 16, 16)] = plsc.load_gather(bx1, [ii])
                gy1[pl.ds(k * 16, 16)] = plsc.load_gather(by1, [ii])
                gx2[pl.ds(k * 16, 16)] = plsc.load_gather(bx2, [ii])
                gy2[pl.ds(k * 16, 16)] = plsc.load_gather(by2, [ii])
                return 0

            lax.fori_loop(0, KP // 16, gb, 0)

            pltpu.sync_copy(fval, ov_hbm.at[c])
            pltpu.sync_copy(gx1, ox1_hbm.at[c])
            pltpu.sync_copy(gy1, oy1_hbm.at[c])
            pltpu.sync_copy(gx2, ox2_hbm.at[c])
            pltpu.sync_copy(gy2, oy2_hbm.at[c])

        def rep_body(rep, _):
            c = wid + 32 * rep

            @pl.when(c < NUM_FG)
            def _():
                do_class(c)

            return 0

        lax.fori_loop(0, 3, rep_body, 0)

    return sc_select


def _pack_matrix():
    j16 = jax.lax.broadcasted_iota(jnp.int32, (KP, NW), 0)
    wl = jax.lax.broadcasted_iota(jnp.int32, (KP, NW), 1)
    pw = jnp.int32(1) << (j16 % 16)
    return jnp.where((j16 // 16) == wl, pw, 0).astype(jnp.float32)


CPG = 4


def _iou_pack_body(x1r_ref, y1r_ref, x2r_ref, y2r_ref,
                   x1c_ref, y1c_ref, x2c_ref, y2c_ref, sp_ref):
    packm = _pack_matrix()
    for q in range(CPG):
        x1r, y1r, x2r, y2r = x1r_ref[q], y1r_ref[q], x2r_ref[q], y2r_ref[q]
        x1c, y1c, x2c, y2c = x1c_ref[q], y1c_ref[q], x2c_ref[q], y2c_ref[q]
        w3 = jnp.maximum(jnp.minimum(x2c, x2r) - jnp.maximum(x1c, x1r), 0.0) * 3.0
        h = jnp.maximum(jnp.minimum(y2c, y2r) - jnp.maximum(y1c, y1r), 0.0)
        inter3 = w3 * h
        arc = (x2c - x1c) * (y2c - y1c) + 5e-10
        arr = (x2r - x1r) * (y2r - y1r) + 5e-10
        s = jnp.where(inter3 > (arc + arr), 1.0, 0.0)
        sp = jnp.dot(s, packm, preferred_element_type=jnp.float32)
        sp_ref[q] = sp.astype(jnp.int32)


def _iou_pack(x1cm, y1cm, x2cm, y2cm):
    row = pl.BlockSpec((CPG, 1, KP), lambda c: (c, 0, 0))
    col = pl.BlockSpec((CPG, KP, 1), lambda c: (c, 0, 0))
    r3 = lambda a: a[:, None, :]
    c3 = lambda a: a[:, :, None]
    return pl.pallas_call(
        _iou_pack_body,
        grid=(NUM_FG // CPG,),
        in_specs=[row, row, row, row, col, col, col, col],
        out_specs=pl.BlockSpec((CPG, KP, NW), lambda c: (c, 0, 0)),
        out_shape=jax.ShapeDtypeStruct((NUM_FG, KP, NW), jnp.int32),
    )(r3(x1cm), r3(y1cm), r3(x2cm), r3(y2cm),
      c3(x1cm), c3(y1cm), c3(x2cm), c3(y2cm))


def _nms_topk_body(sp_ref, vals_ref, x1_ref, y1_ref, x2_ref, y2_ref,
                   x1o_ref, y1o_ref, x2o_ref, y2o_ref, so_ref, scr_ref, rm_ref):
    vals = vals_ref[...]
    validf = jnp.where(vals > SCORE_THRESH, 1.0, 0.0)
    keep = jnp.dot(validf, _pack_matrix(),
                   preferred_element_type=jnp.float32).astype(jnp.int32)

    lane = jax.lax.broadcasted_iota(jnp.int32, (1, NW), 1)
    for w in range(NW):
        later_words = jnp.where(lane > w, jnp.int32(-1), jnp.int32(0))
        for b in range(16):
            g = 16 * w + b
            srow = sp_ref[:, g, :]
            kb = (keep[:, w:w + 1] >> b) & 1
            if b == 15:
                fmask = later_words
            else:
                cur = ((0xFFFF << (b + 1)) & 0xFFFF)
                fmask = later_words | jnp.where(lane == w, jnp.int32(cur), 0)
            keep = keep & ~(srow & fmask & (-kb))

    bit16 = jax.lax.broadcasted_iota(jnp.int32, (1, 16), 1)
    for w in range(NW):
        bits = (keep[:, w:w + 1] >> bit16) & 1
        v = vals[:, 16 * w:16 * (w + 1)]
        scr_ref[:, 16 * w:16 * (w + 1)] = jnp.where(bits == 1, v, -1.0)

    rm_ref[...] = jnp.max(scr_ref[...], axis=1, keepdims=True)
    ci = jax.lax.broadcasted_iota(jnp.int32, (NUM_FG, 1), 0)
    ri = jax.lax.broadcasted_iota(jnp.int32, (1, KP), 1)

    def body(i, _):
        rm = rm_ref[...]
        m = jnp.max(rm)
        cstar = jnp.min(jnp.where(rm == m, ci, BIG))
        row = scr_ref[pl.ds(cstar, 1), :]
        rstar = jnp.min(jnp.where((row == m) & (ri < K), ri, BIG))
        sel = ri == rstar
        x1r = x1_ref[pl.ds(cstar, 1), :]
        y1r = y1_ref[pl.ds(cstar, 1), :]
        x2r = x2_ref[pl.ds(cstar, 1), :]
        y2r = y2_ref[pl.ds(cstar, 1), :]
        x1o_ref[pl.ds(i, 1), :] = jnp.sum(jnp.where(sel, x1r, 0.0)).reshape(1, 1)
        y1o_ref[pl.ds(i, 1), :] = jnp.sum(jnp.where(sel, y1r, 0.0)).reshape(1, 1)
        x2o_ref[pl.ds(i, 1), :] = jnp.sum(jnp.where(sel, x2r, 0.0)).reshape(1, 1)
        y2o_ref[pl.ds(i, 1), :] = jnp.sum(jnp.where(sel, y2r, 0.0)).reshape(1, 1)
        so_ref[pl.ds(i, 1), :] = m.reshape(1, 1)
        newrow = jnp.where(sel, -2.0, row)
        scr_ref[pl.ds(cstar, 1), :] = newrow
        rm_ref[pl.ds(cstar, 1), :] = jnp.max(newrow, axis=1, keepdims=True)
        return 0

    jax.lax.fori_loop(0, DET_PER_IMG, body, 0)


def _nms_topk(s_pack, vals_cm, x1cm, y1cm, x2cm, y2cm):
    o = jax.ShapeDtypeStruct((DET_PER_IMG, 1), jnp.float32)
    return pl.pallas_call(
        _nms_topk_body,
        out_shape=(o, o, o, o, o),
        scratch_shapes=[pltpu.VMEM((NUM_FG, KP), jnp.float32),
                        pltpu.VMEM((NUM_FG, 1), jnp.float32)],
    )(s_pack, vals_cm, x1cm, y1cm, x2cm, y2cm)


def kernel(class_logits, box_regression, proposals):
    probs_cm, x1p, y1p, x2p, y2p = _softmax_decode(
        class_logits, box_regression, proposals)

    vals_cm, x1cm, y1cm, x2cm, y2cm = _make_sc_select()(
        probs_cm, x1p, y1p, x2p, y2p)
    s_pack = _iou_pack(x1cm, y1cm, x2cm, y2cm)
    x1o, y1o, x2o, y2o, so = _nms_topk(s_pack, vals_cm, x1cm, y1cm, x2cm, y2cm)
    return jnp.concatenate([x1o, y1o, x2o, y2o, so], axis=1)
